# Initial kernel scaffold; baseline (speedup 1.0000x reference)
import jax
import jax.numpy as jnp
from jax.experimental import pallas as pl


def kernel(x, feature):
    raise NotImplementedError("write your pallas kernel here")


# speedup vs baseline: 2.3074x; 2.3074x over previous
import functools
import jax
import jax.numpy as jnp
from jax import lax
from jax.experimental import pallas as pl
from jax.experimental.pallas import tpu as pltpu
from jax.experimental.pallas import tpu_sc as plsc

N_PTS = 1048576
FDIM = 16
G = 128
NC, NS, L = 2, 16, 16
NW = NC * NS
PTS_PER_W = N_PTS // NW
B = 128
NBLK = PTS_PER_W // B


def _axis_terms(v):
    xn = (v + 1.0) - 1.0
    ip = ((xn + 1.0) * float(G) - 1.0) * 0.5
    i0 = ip.astype(jnp.int32)
    i0 = jnp.where(i0.astype(jnp.float32) > ip, i0 - 1, i0)
    w = ip - i0.astype(jnp.float32)
    i1 = i0 + 1
    ok0 = (i0 >= 0) & (i0 < G)
    ok1 = (i1 >= 0) & (i1 < G)
    w0 = jnp.where(ok0, 1.0 - w, 0.0)
    w1 = jnp.where(ok1, w, 0.0)
    i0c = jnp.minimum(jnp.maximum(i0, 0), G - 1)
    i1c = jnp.minimum(jnp.maximum(i1, 0), G - 1)
    return i0c, i1c, w0, w1


def _sc_body(table, xq, yq, zq, out, xb, yb, zb, idxb, wb, rows, ob, sem):
    wid = lax.axis_index("s") * NC + lax.axis_index("c")
    base0 = wid * PTS_PER_W

    def block(i, carry):
        base = base0 + i * B
        pltpu.sync_copy(xq.at[pl.ds(base, B)], xb)
        pltpu.sync_copy(yq.at[pl.ds(base, B)], yb)
        pltpu.sync_copy(zq.at[pl.ds(base, B)], zb)

        for j in range(B // L):
            sl = pl.ds(j * L, L)
            x0, x1, wx0, wx1 = _axis_terms(xb[sl])
            y0, y1, wy0, wy1 = _axis_terms(yb[sl])
            z0, z1, wz0, wz1 = _axis_terms(zb[sl])
            zA = z0 * (G * G)
            zB = z1 * (G * G)
            yA = y0 * G
            yB = y1 * G
            pxy = (wx0 * wy0, wx1 * wy0, wx0 * wy1, wx1 * wy1)
            xs = (x0, x1)
            ys = (yA, yB)
            for cz, zterm, wzf in ((0, zA, wz0), (1, zB, wz1)):
                for cy in (0, 1):
                    for cx in (0, 1):
                        c = cz * 4 + cy * 2 + cx
                        idxb[c, sl] = zterm + ys[cy] + xs[cx]
                        wb[c, sl] = pxy[cy * 2 + cx] * wzf

        cps = [
            pltpu.async_copy(table.at[idxb.at[c]], rows.at[c], sem)
            for c in range(8)
        ]
        for cp in cps:
            cp.wait()

        def acc(g, carry2):
            sl = pl.ds(g * L, L)
            wv = [wb[c, sl] for c in range(8)]
            for k in range(L):
                b = g * L + k
                a = wv[0][k] * rows[0, b, :]
                for c in range(1, 8):
                    a = a + wv[c][k] * rows[c, b, :]
                ob[b, :] = a
            return carry2

        lax.fori_loop(0, B // L, acc, 0)
        pltpu.sync_copy(ob, out.at[pl.ds(base, B)])
        return carry

    lax.fori_loop(0, NBLK, block, 0)


@functools.partial(
    pl.kernel,
    out_type=jax.ShapeDtypeStruct((N_PTS, FDIM), jnp.float32),
    mesh=plsc.VectorSubcoreMesh(core_axis_name="c", subcore_axis_name="s"),
    scratch_types=[
        pltpu.VMEM((B,), jnp.float32),
        pltpu.VMEM((B,), jnp.float32),
        pltpu.VMEM((B,), jnp.float32),
        pltpu.VMEM((8, B), jnp.int32),
        pltpu.VMEM((8, B), jnp.float32),
        pltpu.VMEM((8, B, FDIM), jnp.float32),
        pltpu.VMEM((B, FDIM), jnp.float32),
        pltpu.SemaphoreType.DMA,
    ],
    compiler_params=pltpu.CompilerParams(use_tc_tiling_on_sc=False),
)
def _grid_sample_sc(table, xq, yq, zq, out, xb, yb, zb, idxb, wb, rows, ob, sem):
    _sc_body(table, xq, yq, zq, out, xb, yb, zb, idxb, wb, rows, ob, sem)


def kernel(x, feature):
    table = jnp.transpose(feature[0], (1, 2, 3, 0)).reshape(G * G * G, FDIM)
    xq = x[:, 0]
    yq = x[:, 1]
    zq = x[:, 2]
    return _grid_sample_sc(table, xq, yq, zq)
